# baseline (speedup 1.0000x reference)
import functools

import jax
import jax.numpy as jnp
from jax import lax
from jax.experimental import pallas as pl
from jax.experimental.pallas import tpu as pltpu
from jax.experimental.pallas import tpu_sc as plsc

BS, NV, DIM, NC = 8, 4096, 256, 10
NCP = 16
B_SC = 3
B_TC = BS - B_SC
NWORK = 32
RPW = NV // NWORK
KCH = DIM // 16
NEV = 2 * NC
NROW_SC = B_SC * NCP
NROW_TC = B_TC * NCP
NROWS = BS * NCP
ORD_OFF = B_SC * NEV
MLEN = 2 * B_SC * NEV + 16


def _sc_partial_sums(embeddings, meta2):
    mesh = plsc.VectorSubcoreMesh(
        core_axis_name="c", subcore_axis_name="s",
        num_cores=2, num_subcores=16)

    @functools.partial(
        pl.kernel,
        out_type=(
            jax.ShapeDtypeStruct((NWORK, NROW_SC, DIM), jnp.float32),
            jax.ShapeDtypeStruct((NWORK, NROW_SC * 16), jnp.float32),
        ),
        mesh=mesh,
        scratch_types=[
            pltpu.VMEM((MLEN,), jnp.int32),
            pltpu.VMEM((RPW, DIM), jnp.float32),
            pltpu.VMEM((RPW, DIM), jnp.float32),
            pltpu.VMEM((NEV * DIM,), jnp.float32),
            pltpu.VMEM((NEV * 16,), jnp.float32),
            pltpu.VMEM((NROW_SC, DIM), jnp.float32),
            pltpu.VMEM((NROW_SC * 16,), jnp.float32),
            pltpu.SMEM((2 * NEV,), jnp.int32),
            pltpu.SemaphoreType.DMA,
            pltpu.SemaphoreType.DMA,
        ],
    )
    def k(emb_hbm, meta2_hbm, sum_out, sq_out,
          meta_v, chunk0_v, chunk1_v, snap_v, snapsq_v, acc_v, sq_v,
          ev_s, sem0, sem1):
        wid = lax.axis_index("s") * 2 + lax.axis_index("c")
        base = wid * RPW
        pltpu.sync_copy(meta2_hbm.at[wid], meta_v)
        chunks = (chunk0_v, chunk1_v)
        sems = (sem0, sem1)

        zv = jnp.zeros((16,), jnp.float32)

        def zero_body(i, _):
            for kk in range(KCH):
                acc_v[i, pl.ds(kk * 16, 16)] = zv
            sq_v[pl.ds(i * 16, 16)] = zv
            return 0

        lax.fori_loop(0, NROW_SC, zero_body, 0)

        copies = [None, None]
        copies[0] = pltpu.async_copy(
            emb_hbm.at[0, pl.ds(base, RPW), :], chunks[0], sems[0])
        for b in range(B_SC):
            cur = b % 2
            copies[cur].wait()
            if b + 1 < B_SC:
                nxt = (b + 1) % 2
                copies[nxt] = pltpu.async_copy(
                    emb_hbm.at[b + 1, pl.ds(base, RPW), :], chunks[nxt],
                    sems[nxt])
            chunk_v = chunks[cur]

            for t in range(NEV):
                ev_s[t] = meta_v[pl.ds(b * NEV + t, 16)][0]
                ev_s[NEV + t] = meta_v[pl.ds(ORD_OFF + b * NEV + t, 16)][0]

            def ev_body(t, carry, chunk_v=chunk_v):
                accs, sqs, rprev = carry
                r1 = ev_s[t]

                def row_body(p, rc):
                    raccs, rsqs = rc
                    na = []
                    nq = list(rsqs)
                    for kk in range(KCH):
                        v = chunk_v[p, pl.ds(kk * 16, 16)]
                        na.append(raccs[kk] + v)
                        nq[kk % 4] = nq[kk % 4] + v * v
                    return (tuple(na), tuple(nq))

                accs, sqs = lax.fori_loop(rprev, r1, row_body, (accs, sqs))
                slot = ev_s[NEV + t]
                for kk in range(KCH):
                    snap_v[pl.ds(slot * DIM + kk * 16, 16)] = accs[kk]
                snapsq_v[pl.ds(slot * 16, 16)] = (
                    (sqs[0] + sqs[1]) + (sqs[2] + sqs[3]))
                return (accs, sqs, r1)

            z = tuple(jnp.zeros((16,), jnp.float32) for _ in range(KCH))
            z4 = tuple(jnp.zeros((16,), jnp.float32) for _ in range(4))
            r0 = ev_s[0]
            lax.fori_loop(0, NEV, ev_body, (z, z4, r0))

            def cluster_body(j, _, b=b):
                seg = b * NCP + j
                for kk in range(KCH):
                    acc_v[seg, pl.ds(kk * 16, 16)] = (
                        snap_v[pl.ds((NC + j) * DIM + kk * 16, 16)]
                        - snap_v[pl.ds(j * DIM + kk * 16, 16)])
                sq_v[pl.ds(seg * 16, 16)] = (
                    snapsq_v[pl.ds((NC + j) * 16, 16)]
                    - snapsq_v[pl.ds(j * 16, 16)])
                return 0

            lax.fori_loop(0, NC, cluster_body, 0, unroll=2)

        pltpu.sync_copy(acc_v, sum_out.at[wid])
        pltpu.sync_copy(sq_v, sq_out.at[wid])

    return k(embeddings, meta2)


def _tc_partial_sums(embeddings, se_tc):
    def body(se_ref, e_ref, s_ref, q_ref):
        e = e_ref[0]
        se = se_ref[0]
        bi = lax.broadcasted_iota(jnp.int32, (NCP, NV), 1).astype(jnp.float32)
        m = ((bi >= se[:, 0:1]) & (bi < se[:, 1:2])).astype(jnp.float32)
        s_ref[0] = jnp.dot(m, e, preferred_element_type=jnp.float32)
        q_ref[0] = jnp.dot(m, e * e, preferred_element_type=jnp.float32)

    return pl.pallas_call(
        body,
        grid=(B_TC,),
        in_specs=[
            pl.BlockSpec((1, NCP, 2), lambda b: (b, 0, 0)),
            pl.BlockSpec((1, NV, DIM), lambda b: (b + B_SC, 0, 0)),
        ],
        out_specs=[
            pl.BlockSpec((1, NCP, DIM), lambda b: (b, 0, 0)),
            pl.BlockSpec((1, NCP, DIM), lambda b: (b, 0, 0)),
        ],
        out_shape=[
            jax.ShapeDtypeStruct((B_TC, NCP, DIM), jnp.float32),
            jax.ShapeDtypeStruct((B_TC, NCP, DIM), jnp.float32),
        ],
    )(se_tc, embeddings)


def _tc_finish(sc_sum, sc_sq, tc_sum, tc_sq, nf):
    def body(scs_ref, scq_ref, tcs_ref, tcq_ref, nf_ref, out_ref):
        s_sc = scs_ref[0]
        q_sc = scq_ref[0]
        for w in range(1, NWORK):
            s_sc = s_sc + scs_ref[w]
            q_sc = q_sc + scq_ref[w]
        s = jnp.concatenate([s_sc, tcs_ref[...]], axis=0)
        sumsq = jnp.concatenate([
            jnp.sum(q_sc, axis=1, keepdims=True),
            jnp.sum(tcq_ref[...], axis=1, keepdims=True),
        ], axis=0)
        nf1 = nf_ref[...]
        mean = s / nf1
        msq = jnp.sum(mean * mean, axis=1, keepdims=True)
        var_total = jnp.sum((sumsq - nf1 * msq) / (nf1 - 1.0))
        norm = jnp.sqrt(msq)
        m = mean / jnp.maximum(norm, 1e-12)
        g = lax.dot_general(m, m, (((1,), (1,)), ((), ())))
        row = lax.broadcasted_iota(jnp.int32, (NROWS, NROWS), 0)
        col = lax.broadcasted_iota(jnp.int32, (NROWS, NROWS), 1)
        same = ((row // NCP) == (col // NCP)) & (row != col)
        sum_g = jnp.sum(jnp.where(same, g, 0.0))
        pairs_per_batch = NC * (NC - 1) // 2
        loss = 0.1 * (float(BS * pairs_per_batch) + 0.5 * sum_g) + var_total
        out_ref[...] = jnp.reshape(loss, (1, 1))

    out = pl.pallas_call(
        body,
        out_shape=jax.ShapeDtypeStruct((1, 1), jnp.float32),
    )(sc_sum, sc_sq, tc_sum, tc_sq, nf)
    return out.reshape(1)


def kernel(embeddings, cluster_sizes):
    cs = cluster_sizes.astype(jnp.int32)
    starts = jnp.concatenate(
        [jnp.zeros((BS, 1), jnp.int32), cs[:, :-1]], axis=1)
    ends = starts + cs

    base = (jnp.arange(NWORK, dtype=jnp.int32) * RPW)[:, None, None]
    lo = jnp.clip(starts[None, :B_SC] - base, 0, RPW)
    hi = jnp.clip(ends[None, :B_SC] - base, 0, RPW)
    ev = jnp.concatenate([lo, hi], axis=2)
    tid = jnp.arange(NEV, dtype=jnp.int32)
    before = (ev[..., :, None] > ev[..., None, :]) | (
        (ev[..., :, None] == ev[..., None, :])
        & (tid[:, None] > tid[None, :]))
    rank = jnp.sum(before.astype(jnp.int32), axis=3)
    onehot = (rank[..., :, None] == tid[None, :]).astype(jnp.int32)
    ev_sorted = jnp.sum(onehot * ev[..., :, None], axis=2)
    order = jnp.sum(onehot * tid[:, None], axis=2).astype(jnp.int32)
    meta2 = jnp.concatenate([
        ev_sorted.reshape(NWORK, B_SC * NEV),
        order.reshape(NWORK, B_SC * NEV),
        jnp.zeros((NWORK, 16), jnp.int32),
    ], axis=1)

    pad_se = jnp.zeros((B_TC, NCP - NC), jnp.int32)
    se_tc = jnp.stack([
        jnp.concatenate([starts[B_SC:], pad_se], axis=1),
        jnp.concatenate([ends[B_SC:], pad_se], axis=1),
    ], axis=2).astype(jnp.float32)

    cs_pad = jnp.concatenate(
        [cs.astype(jnp.float32), jnp.full((BS, NCP - NC), 2.0)], axis=1)
    nf = cs_pad.reshape(NROWS, 1)

    tc_sum, tc_sq = _tc_partial_sums(embeddings, se_tc)
    sc_sum, sc_sq = _sc_partial_sums(embeddings, meta2)
    return _tc_finish(sc_sum, sc_sq.reshape(NWORK, NROW_SC, 16),
                      tc_sum.reshape(NROW_TC, DIM),
                      tc_sq.reshape(NROW_TC, DIM), nf)


# speedup vs baseline: 1.0383x; 1.0383x over previous
import functools

import jax
import jax.numpy as jnp
from jax import lax
from jax.experimental import pallas as pl
from jax.experimental.pallas import tpu as pltpu
from jax.experimental.pallas import tpu_sc as plsc

BS, NV, DIM, NC = 8, 4096, 256, 10
NCP = 16
B_SC = 2
B_TC = BS - B_SC
NWORK = 32
RPW = NV // NWORK
KCH = DIM // 16
NEV = 2 * NC
NROW_SC = B_SC * NCP
NROW_TC = B_TC * NCP
NROWS = BS * NCP
ORD_OFF = B_SC * NEV
MLEN = 2 * B_SC * NEV + 16


def _sc_partial_sums(embeddings, meta2):
    mesh = plsc.VectorSubcoreMesh(
        core_axis_name="c", subcore_axis_name="s",
        num_cores=2, num_subcores=16)

    @functools.partial(
        pl.kernel,
        out_type=(
            jax.ShapeDtypeStruct((NWORK, NROW_SC, DIM), jnp.float32),
            jax.ShapeDtypeStruct((NWORK, NROW_SC * 16), jnp.float32),
        ),
        mesh=mesh,
        scratch_types=[
            pltpu.VMEM((MLEN,), jnp.int32),
            pltpu.VMEM((RPW, DIM), jnp.float32),
            pltpu.VMEM((RPW, DIM), jnp.float32),
            pltpu.VMEM((NEV * DIM,), jnp.float32),
            pltpu.VMEM((NEV * 16,), jnp.float32),
            pltpu.VMEM((NROW_SC, DIM), jnp.float32),
            pltpu.VMEM((NROW_SC * 16,), jnp.float32),
            pltpu.SMEM((2 * NEV,), jnp.int32),
            pltpu.SemaphoreType.DMA,
            pltpu.SemaphoreType.DMA,
        ],
    )
    def k(emb_hbm, meta2_hbm, sum_out, sq_out,
          meta_v, chunk0_v, chunk1_v, snap_v, snapsq_v, acc_v, sq_v,
          ev_s, sem0, sem1):
        wid = lax.axis_index("s") * 2 + lax.axis_index("c")
        base = wid * RPW
        pltpu.sync_copy(meta2_hbm.at[wid], meta_v)
        chunks = (chunk0_v, chunk1_v)
        sems = (sem0, sem1)

        zv = jnp.zeros((16,), jnp.float32)

        def zero_body(i, _):
            for kk in range(KCH):
                acc_v[i, pl.ds(kk * 16, 16)] = zv
            sq_v[pl.ds(i * 16, 16)] = zv
            return 0

        lax.fori_loop(0, NROW_SC, zero_body, 0)

        copies = [None, None]
        copies[0] = pltpu.async_copy(
            emb_hbm.at[0, pl.ds(base, RPW), :], chunks[0], sems[0])
        for b in range(B_SC):
            cur = b % 2
            copies[cur].wait()
            if b + 1 < B_SC:
                nxt = (b + 1) % 2
                copies[nxt] = pltpu.async_copy(
                    emb_hbm.at[b + 1, pl.ds(base, RPW), :], chunks[nxt],
                    sems[nxt])
            chunk_v = chunks[cur]

            for t in range(NEV):
                ev_s[t] = meta_v[pl.ds(b * NEV + t, 16)][0]
                ev_s[NEV + t] = meta_v[pl.ds(ORD_OFF + b * NEV + t, 16)][0]

            def ev_body(t, carry, chunk_v=chunk_v):
                accs, sqs, rprev = carry
                r1 = ev_s[t]

                def row_body(p, rc):
                    raccs, rsqs = rc
                    na = []
                    nq = list(rsqs)
                    for kk in range(KCH):
                        v = chunk_v[p, pl.ds(kk * 16, 16)]
                        na.append(raccs[kk] + v)
                        nq[kk % 4] = nq[kk % 4] + v * v
                    return (tuple(na), tuple(nq))

                accs, sqs = lax.fori_loop(rprev, r1, row_body, (accs, sqs))
                slot = ev_s[NEV + t]
                for kk in range(KCH):
                    snap_v[pl.ds(slot * DIM + kk * 16, 16)] = accs[kk]
                snapsq_v[pl.ds(slot * 16, 16)] = (
                    (sqs[0] + sqs[1]) + (sqs[2] + sqs[3]))
                return (accs, sqs, r1)

            z = tuple(jnp.zeros((16,), jnp.float32) for _ in range(KCH))
            z4 = tuple(jnp.zeros((16,), jnp.float32) for _ in range(4))
            r0 = ev_s[0]
            lax.fori_loop(0, NEV, ev_body, (z, z4, r0))

            def cluster_body(j, _, b=b):
                seg = b * NCP + j
                for kk in range(KCH):
                    acc_v[seg, pl.ds(kk * 16, 16)] = (
                        snap_v[pl.ds((NC + j) * DIM + kk * 16, 16)]
                        - snap_v[pl.ds(j * DIM + kk * 16, 16)])
                sq_v[pl.ds(seg * 16, 16)] = (
                    snapsq_v[pl.ds((NC + j) * 16, 16)]
                    - snapsq_v[pl.ds(j * 16, 16)])
                return 0

            lax.fori_loop(0, NC, cluster_body, 0, unroll=2)

        pltpu.sync_copy(acc_v, sum_out.at[wid])
        pltpu.sync_copy(sq_v, sq_out.at[wid])

    return k(embeddings, meta2)


def _tc_partial_sums(embeddings, se_tc):
    def body(se_ref, e_ref, s_ref, q_ref):
        e = e_ref[0]
        se = se_ref[0]
        bi = lax.broadcasted_iota(jnp.int32, (NCP, NV), 1).astype(jnp.float32)
        m = ((bi >= se[:, 0:1]) & (bi < se[:, 1:2])).astype(jnp.float32)
        s_ref[0] = jnp.dot(m, e, preferred_element_type=jnp.float32)
        q_ref[0] = jnp.dot(m, e * e, preferred_element_type=jnp.float32)

    return pl.pallas_call(
        body,
        grid=(B_TC,),
        in_specs=[
            pl.BlockSpec((1, NCP, 2), lambda b: (b, 0, 0)),
            pl.BlockSpec((1, NV, DIM), lambda b: (b + B_SC, 0, 0)),
        ],
        out_specs=[
            pl.BlockSpec((1, NCP, DIM), lambda b: (b, 0, 0)),
            pl.BlockSpec((1, NCP, DIM), lambda b: (b, 0, 0)),
        ],
        out_shape=[
            jax.ShapeDtypeStruct((B_TC, NCP, DIM), jnp.float32),
            jax.ShapeDtypeStruct((B_TC, NCP, DIM), jnp.float32),
        ],
    )(se_tc, embeddings)


def _tc_finish(sc_sum, sc_sq, tc_sum, tc_sq, nf):
    def body(scs_ref, scq_ref, tcs_ref, tcq_ref, nf_ref, out_ref):
        s_sc = scs_ref[0]
        q_sc = scq_ref[0]
        for w in range(1, NWORK):
            s_sc = s_sc + scs_ref[w]
            q_sc = q_sc + scq_ref[w]
        s = jnp.concatenate([s_sc, tcs_ref[...]], axis=0)
        sumsq = jnp.concatenate([
            jnp.sum(q_sc, axis=1, keepdims=True),
            jnp.sum(tcq_ref[...], axis=1, keepdims=True),
        ], axis=0)
        nf1 = nf_ref[...]
        mean = s / nf1
        msq = jnp.sum(mean * mean, axis=1, keepdims=True)
        var_total = jnp.sum((sumsq - nf1 * msq) / (nf1 - 1.0))
        norm = jnp.sqrt(msq)
        m = mean / jnp.maximum(norm, 1e-12)
        g = lax.dot_general(m, m, (((1,), (1,)), ((), ())))
        row = lax.broadcasted_iota(jnp.int32, (NROWS, NROWS), 0)
        col = lax.broadcasted_iota(jnp.int32, (NROWS, NROWS), 1)
        same = ((row // NCP) == (col // NCP)) & (row != col)
        sum_g = jnp.sum(jnp.where(same, g, 0.0))
        pairs_per_batch = NC * (NC - 1) // 2
        loss = 0.1 * (float(BS * pairs_per_batch) + 0.5 * sum_g) + var_total
        out_ref[...] = jnp.reshape(loss, (1, 1))

    out = pl.pallas_call(
        body,
        out_shape=jax.ShapeDtypeStruct((1, 1), jnp.float32),
    )(sc_sum, sc_sq, tc_sum, tc_sq, nf)
    return out.reshape(1)


def kernel(embeddings, cluster_sizes):
    cs = cluster_sizes.astype(jnp.int32)
    starts = jnp.concatenate(
        [jnp.zeros((BS, 1), jnp.int32), cs[:, :-1]], axis=1)
    ends = starts + cs

    base = (jnp.arange(NWORK, dtype=jnp.int32) * RPW)[:, None, None]
    lo = jnp.clip(starts[None, :B_SC] - base, 0, RPW)
    hi = jnp.clip(ends[None, :B_SC] - base, 0, RPW)
    ev = jnp.concatenate([lo, hi], axis=2)
    tid = jnp.arange(NEV, dtype=jnp.int32)
    before = (ev[..., :, None] > ev[..., None, :]) | (
        (ev[..., :, None] == ev[..., None, :])
        & (tid[:, None] > tid[None, :]))
    rank = jnp.sum(before.astype(jnp.int32), axis=3)
    onehot = (rank[..., :, None] == tid[None, :]).astype(jnp.int32)
    ev_sorted = jnp.sum(onehot * ev[..., :, None], axis=2)
    order = jnp.sum(onehot * tid[:, None], axis=2).astype(jnp.int32)
    meta2 = jnp.concatenate([
        ev_sorted.reshape(NWORK, B_SC * NEV),
        order.reshape(NWORK, B_SC * NEV),
        jnp.zeros((NWORK, 16), jnp.int32),
    ], axis=1)

    pad_se = jnp.zeros((B_TC, NCP - NC), jnp.int32)
    se_tc = jnp.stack([
        jnp.concatenate([starts[B_SC:], pad_se], axis=1),
        jnp.concatenate([ends[B_SC:], pad_se], axis=1),
    ], axis=2).astype(jnp.float32)

    cs_pad = jnp.concatenate(
        [cs.astype(jnp.float32), jnp.full((BS, NCP - NC), 2.0)], axis=1)
    nf = cs_pad.reshape(NROWS, 1)

    tc_sum, tc_sq = _tc_partial_sums(embeddings, se_tc)
    sc_sum, sc_sq = _sc_partial_sums(embeddings, meta2)
    return _tc_finish(sc_sum, sc_sq.reshape(NWORK, NROW_SC, 16),
                      tc_sum.reshape(NROW_TC, DIM),
                      tc_sq.reshape(NROW_TC, DIM), nf)
